# baseline (speedup 1.0000x reference)
import functools

import jax
import jax.numpy as jnp
from jax import lax
from jax.experimental import pallas as pl
from jax.experimental.pallas import tpu as pltpu
from jax.experimental.pallas import tpu_sc as plsc

D = 56
HW = D * D
M = 2048
C = 384
DOUT = 256
R = 7
T = 2 * R + 1
NW = 32
QPW = M // NW
BIGF = 3.0e38


def _bf16_rne(x_f32):
    c = jnp.float32(65537.0)
    y = x_f32 * c
    z = y - x_f32
    return y - z


def _tc_pixproj(img_ref, w1_ref, p_ref):
    img = img_ref[0].astype(jnp.bfloat16)
    out = lax.dot_general(
        img, w1_ref[...],
        dimension_numbers=(((0,), (1,)), ((), ())),
        preferred_element_type=jnp.float32,
    )
    p_ref[...] = out


def _tc_xw2b(x_ref, w2t_ref, b_ref, o_ref):
    o_ref[...] = (
        jnp.dot(x_ref[...], w2t_ref[...], preferred_element_type=jnp.float32)
        + b_ref[...]
    )


def _sc_select(posx_hbm, posy_hbm, batch_hbm, cxb_hbm, qx_hbm,
               wn_hbm, ix_hbm,
               posx_v, posy_v, batch_v, cxb_v, qx_v,
               axbuf, aybuf, qxbuf, qybuf, penx, peny, p2buf,
               wbuf, ibuf):
    wid = lax.axis_index("s") * 2 + lax.axis_index("c")
    base = wid * QPW

    pltpu.sync_copy(posx_hbm.at[pl.ds(base, QPW)], posx_v)
    pltpu.sync_copy(posy_hbm.at[pl.ds(base, QPW)], posy_v)
    pltpu.sync_copy(batch_hbm.at[pl.ds(base, QPW)], batch_v)
    pltpu.sync_copy(cxb_hbm, cxb_v)
    pltpu.sync_copy(qx_hbm, qx_v)

    for g in range(QPW // 16):
        px = posx_v[pl.ds(g * 16, 16)]
        py = posy_v[pl.ds(g * 16, 16)]
        bq = batch_v[pl.ds(g * 16, 16)]
        fD = jnp.float32(D)
        jstar = jnp.clip((px * fD).astype(jnp.int32), 0, D - 1)
        istar = jnp.clip((py * fD).astype(jnp.int32), 0, D - 1)
        p2buf[pl.ds(0, 16)] = px * px
        p2buf[pl.ds(16, 16)] = py * py
        p2 = p2buf[pl.ds(0, 16)] + p2buf[pl.ds(16, 16)]
        pxb = _bf16_rne(px)
        pyb = _bf16_rne(py)

        for t in range(T):
            jj = jstar + (t - R)
            jvalid = (jj >= 0) & (jj <= D - 1)
            jc = jnp.clip(jj, 0, D - 1)
            cxv = plsc.load_gather(cxb_v, [jc])
            qxv = plsc.load_gather(qx_v, [jc])
            axbuf[pl.ds(t * 16, 16)] = pxb * cxv
            qxbuf[pl.ds(t * 16, 16)] = qxv
            penx[pl.ds(t * 16, 16)] = jnp.where(jvalid, 0.0, BIGF)

            ii = istar + (t - R)
            ivalid = (ii >= 0) & (ii <= D - 1)
            ic = jnp.clip(ii, 0, D - 1)
            cyv = plsc.load_gather(cxb_v, [ic])
            qyv = plsc.load_gather(qx_v, [ic])
            aybuf[pl.ds(t * 16, 16)] = pyb * cyv
            qybuf[pl.ds(t * 16, 16)] = qyv
            peny[pl.ds(t * 16, 16)] = jnp.where(ivalid, 0.0, BIGF)

        rowbase0 = bq * HW + (istar - R) * D + (jstar - R)

        def srow(s, carry):
            b1, b2, b3, i1, i2, i3 = carry
            ay = aybuf[pl.ds(s * 16, 16)]
            qy = qybuf[pl.ds(s * 16, 16)]
            pys = peny[pl.ds(s * 16, 16)]
            ibase = rowbase0 + s * D
            for t in range(T):
                ax = axbuf[pl.ds(t * 16, 16)]
                qx_t = qxbuf[pl.ds(t * 16, 16)]
                pxs = penx[pl.ds(t * 16, 16)]
                q2 = qx_t + qy
                t1 = p2 + q2
                dsum = ax + ay
                d2 = t1 - 2.0 * dsum
                d2 = jnp.maximum(d2, 0.0)
                d2 = d2 + pxs + pys
                ic = ibase + t
                lt1 = d2 < b1
                lt2 = d2 < b2
                lt3 = d2 < b3
                nb3 = jnp.where(lt2, b2, jnp.where(lt3, d2, b3))
                ni3 = jnp.where(lt2, i2, jnp.where(lt3, ic, i3))
                nb2 = jnp.where(lt1, b1, jnp.where(lt2, d2, b2))
                ni2 = jnp.where(lt1, i1, jnp.where(lt2, ic, i2))
                b1 = jnp.where(lt1, d2, b1)
                i1 = jnp.where(lt1, ic, i1)
                b2, b3, i2, i3 = nb2, nb3, ni2, ni3
            return b1, b2, b3, i1, i2, i3

        big = jnp.full((16,), BIGF, jnp.float32)
        zi = jnp.zeros((16,), jnp.int32)
        b1, b2, b3, i1, i2, i3 = lax.fori_loop(
            0, T, srow, (big, big, big, zi, zi, zi), unroll=False)

        w1 = 1.0 / jnp.maximum(b1, 1e-16)
        w2 = 1.0 / jnp.maximum(b2, 1e-16)
        w3 = 1.0 / jnp.maximum(b3, 1e-16)
        ws = w1 + w2 + w3
        for c, (wv, iv) in enumerate(((w1, i1), (w2, i2), (w3, i3))):
            off = c * QPW + g * 16
            wbuf[pl.ds(off, 16)] = wv / ws
            ibuf[pl.ds(off, 16)] = iv

    for c in range(3):
        pltpu.sync_copy(wbuf.at[pl.ds(c * QPW, QPW)],
                        wn_hbm.at[c, pl.ds(base, QPW)])
        pltpu.sync_copy(ibuf.at[pl.ds(c * QPW, QPW)],
                        ix_hbm.at[c, pl.ds(base, QPW)])


def _sc_combine(p_hbm, xw2b_hbm, wn_hbm, ix_hbm, out_hbm,
                wbuf, idx3, rows_v, out_v, sem0, sem1, sem2):
    wid = lax.axis_index("s") * 2 + lax.axis_index("c")
    base = wid * QPW

    for c in range(3):
        pltpu.sync_copy(ix_hbm.at[c, pl.ds(base, QPW)], idx3.at[c])
    sems = (sem0, sem1, sem2)
    cps = [pltpu.async_copy(p_hbm.at[idx3.at[c]],
                            rows_v.at[pl.ds(c * QPW, QPW)], sems[c])
           for c in range(3)]
    pltpu.sync_copy(xw2b_hbm.at[pl.ds(base, QPW)], out_v)
    for c in range(3):
        pltpu.sync_copy(wn_hbm.at[c, pl.ds(base, QPW)],
                        wbuf.at[pl.ds(c * QPW, QPW)])
    for cp in cps:
        cp.wait()

    def qacc(q, _):
        qv = jnp.zeros((16,), jnp.int32) + q
        w0 = plsc.load_gather(wbuf, [qv])
        w1_ = plsc.load_gather(wbuf, [qv + QPW])
        w2_ = plsc.load_gather(wbuf, [qv + 2 * QPW])
        for v in range(DOUT // 16):
            sl = pl.ds(v * 16, 16)
            acc = out_v[q, sl]
            acc = acc + w0 * rows_v[q, sl]
            acc = acc + w1_ * rows_v[QPW + q, sl]
            acc = acc + w2_ * rows_v[2 * QPW + q, sl]
            out_v[q, sl] = acc
        return 0

    lax.fori_loop(0, QPW, qacc, 0, unroll=4)

    pltpu.sync_copy(out_v, out_hbm.at[pl.ds(base, QPW)])


def kernel(images, x, pos, batch, W, b):
    Bq = images.shape[0]
    f32 = jnp.float32
    bf16 = jnp.bfloat16

    W1 = W[:, :C].astype(bf16)
    W2T = jnp.transpose(W[:, C:]).astype(bf16)
    centers = jnp.linspace(1.0 / (D * 2), 1.0 - 1.0 / (D * 2), D).astype(f32)
    cxb = jnp.pad(centers.astype(bf16).astype(f32), (0, 8))
    qx = jnp.pad(centers * centers, (0, 8))
    posx = pos[:, 0]
    posy = pos[:, 1]
    batch_i = batch.astype(jnp.int32)

    mesh = plsc.VectorSubcoreMesh(core_axis_name="c", subcore_axis_name="s")
    scparams = pltpu.CompilerParams(needs_layout_passes=False)

    sel = functools.partial(
        pl.kernel,
        mesh=mesh,
        compiler_params=scparams,
        out_type=(jax.ShapeDtypeStruct((3, M), f32),
                  jax.ShapeDtypeStruct((3, M), jnp.int32)),
        scratch_types=[
            pltpu.VMEM((QPW,), f32),
            pltpu.VMEM((QPW,), f32),
            pltpu.VMEM((QPW,), jnp.int32),
            pltpu.VMEM((64,), f32),
            pltpu.VMEM((64,), f32),
            pltpu.VMEM((T * 16,), f32),
            pltpu.VMEM((T * 16,), f32),
            pltpu.VMEM((T * 16,), f32),
            pltpu.VMEM((T * 16,), f32),
            pltpu.VMEM((T * 16,), f32),
            pltpu.VMEM((T * 16,), f32),
            pltpu.VMEM((32,), f32),
            pltpu.VMEM((3 * QPW,), f32),
            pltpu.VMEM((3 * QPW,), jnp.int32),
        ],
    )(_sc_select)
    wn3, ix3 = sel(posx, posy, batch_i, cxb, qx)

    imgs = images.reshape(Bq, C, HW)
    P = pl.pallas_call(
        _tc_pixproj,
        grid=(Bq,),
        in_specs=[
            pl.BlockSpec((1, C, HW), lambda i: (i, 0, 0)),
            pl.BlockSpec((DOUT, C), lambda i: (0, 0)),
        ],
        out_specs=pl.BlockSpec((HW, DOUT), lambda i: (i, 0)),
        out_shape=jax.ShapeDtypeStruct((Bq * HW, DOUT), f32),
    )(imgs, W1)

    xw2b = pl.pallas_call(
        _tc_xw2b,
        in_specs=[
            pl.BlockSpec(x.shape, lambda: (0, 0)),
            pl.BlockSpec(W2T.shape, lambda: (0, 0)),
            pl.BlockSpec((1, DOUT), lambda: (0, 0)),
        ],
        out_specs=pl.BlockSpec((M, DOUT), lambda: (0, 0)),
        out_shape=jax.ShapeDtypeStruct((M, DOUT), f32),
    )(x.astype(bf16), W2T, b.reshape(1, DOUT))

    comb = functools.partial(
        pl.kernel,
        mesh=mesh,
        compiler_params=scparams,
        out_type=jax.ShapeDtypeStruct((M, DOUT), f32),
        scratch_types=[
            pltpu.VMEM((3 * QPW,), f32),
            pltpu.VMEM((3, QPW), jnp.int32),
            pltpu.VMEM((3 * QPW, DOUT), f32),
            pltpu.VMEM((QPW, DOUT), f32),
            pltpu.SemaphoreType.DMA,
            pltpu.SemaphoreType.DMA,
            pltpu.SemaphoreType.DMA,
        ],
    )(_sc_combine)
    out = comb(P, xw2b, wn3, ix3)

    return (out, pos, batch)


# speedup vs baseline: 1.0688x; 1.0688x over previous
import functools

import jax
import jax.numpy as jnp
from jax import lax
from jax.experimental import pallas as pl
from jax.experimental.pallas import tpu as pltpu
from jax.experimental.pallas import tpu_sc as plsc

D = 56
HW = D * D
M = 2048
C = 384
DOUT = 256
R = 7
T = 2 * R + 1
NW = 32
QPW = M // NW
BIGF = 3.0e38


def _bf16_rne(x_f32):
    c = jnp.float32(65537.0)
    y = x_f32 * c
    z = y - x_f32
    return y - z


def _tc_pixproj(img_ref, w1_ref, p_ref):
    img = img_ref[0].astype(jnp.bfloat16)
    out = lax.dot_general(
        img, w1_ref[...],
        dimension_numbers=(((0,), (1,)), ((), ())),
        preferred_element_type=jnp.float32,
    )
    p_ref[...] = out


def _tc_xw2b(x_ref, w2t_ref, b_ref, o_ref):
    o_ref[...] = (
        jnp.dot(x_ref[...], w2t_ref[...], preferred_element_type=jnp.float32)
        + b_ref[...]
    )


def _sc_select(posx_hbm, posy_hbm, batch_hbm, cxb_hbm, qx_hbm,
               wn_hbm, ix_hbm,
               posx_v, posy_v, batch_v, cxb_v, qx_v,
               axbuf, aybuf, qxbuf, qybuf, penx, peny, p2buf,
               wbuf, ibuf):
    wid = lax.axis_index("s") * 2 + lax.axis_index("c")
    base = wid * QPW

    pltpu.sync_copy(posx_hbm.at[pl.ds(base, QPW)], posx_v)
    pltpu.sync_copy(posy_hbm.at[pl.ds(base, QPW)], posy_v)
    pltpu.sync_copy(batch_hbm.at[pl.ds(base, QPW)], batch_v)
    pltpu.sync_copy(cxb_hbm, cxb_v)
    pltpu.sync_copy(qx_hbm, qx_v)

    for g in range(QPW // 16):
        px = posx_v[pl.ds(g * 16, 16)]
        py = posy_v[pl.ds(g * 16, 16)]
        bq = batch_v[pl.ds(g * 16, 16)]
        fD = jnp.float32(D)
        jstar = jnp.clip((px * fD).astype(jnp.int32), 0, D - 1)
        istar = jnp.clip((py * fD).astype(jnp.int32), 0, D - 1)
        p2buf[pl.ds(0, 16)] = px * px
        p2buf[pl.ds(16, 16)] = py * py
        p2 = p2buf[pl.ds(0, 16)] + p2buf[pl.ds(16, 16)]
        pxb = _bf16_rne(px)
        pyb = _bf16_rne(py)

        for t in range(T):
            jj = jstar + (t - R)
            jvalid = (jj >= 0) & (jj <= D - 1)
            jc = jnp.clip(jj, 0, D - 1)
            cxv = plsc.load_gather(cxb_v, [jc])
            qxv = plsc.load_gather(qx_v, [jc])
            axbuf[pl.ds(t * 16, 16)] = pxb * cxv
            qxbuf[pl.ds(t * 16, 16)] = qxv
            penx[pl.ds(t * 16, 16)] = jnp.where(jvalid, 0.0, BIGF)

            ii = istar + (t - R)
            ivalid = (ii >= 0) & (ii <= D - 1)
            ic = jnp.clip(ii, 0, D - 1)
            cyv = plsc.load_gather(cxb_v, [ic])
            qyv = plsc.load_gather(qx_v, [ic])
            aybuf[pl.ds(t * 16, 16)] = pyb * cyv
            qybuf[pl.ds(t * 16, 16)] = qyv
            peny[pl.ds(t * 16, 16)] = jnp.where(ivalid, 0.0, BIGF)

        rowbase0 = bq * HW + (istar - R) * D + (jstar - R)

        def srow(s, carry):
            b1, b2, b3, i1, i2, i3 = carry
            ay = aybuf[pl.ds(s * 16, 16)]
            qy = qybuf[pl.ds(s * 16, 16)]
            pys = peny[pl.ds(s * 16, 16)]
            ibase = rowbase0 + s * D
            for t in range(T):
                ax = axbuf[pl.ds(t * 16, 16)]
                qx_t = qxbuf[pl.ds(t * 16, 16)]
                pxs = penx[pl.ds(t * 16, 16)]
                q2 = qx_t + qy
                t1 = p2 + q2
                dsum = ax + ay
                d2 = t1 - 2.0 * dsum
                d2 = jnp.maximum(d2, 0.0)
                d2 = d2 + pxs + pys
                ic = ibase + t
                lt1 = d2 < b1
                lt2 = d2 < b2
                lt3 = d2 < b3
                nb3 = jnp.where(lt2, b2, jnp.where(lt3, d2, b3))
                ni3 = jnp.where(lt2, i2, jnp.where(lt3, ic, i3))
                nb2 = jnp.where(lt1, b1, jnp.where(lt2, d2, b2))
                ni2 = jnp.where(lt1, i1, jnp.where(lt2, ic, i2))
                b1 = jnp.where(lt1, d2, b1)
                i1 = jnp.where(lt1, ic, i1)
                b2, b3, i2, i3 = nb2, nb3, ni2, ni3
            return b1, b2, b3, i1, i2, i3

        big = jnp.full((16,), BIGF, jnp.float32)
        zi = jnp.zeros((16,), jnp.int32)
        b1, b2, b3, i1, i2, i3 = lax.fori_loop(
            0, T, srow, (big, big, big, zi, zi, zi), unroll=False)

        w1 = 1.0 / jnp.maximum(b1, 1e-16)
        w2 = 1.0 / jnp.maximum(b2, 1e-16)
        w3 = 1.0 / jnp.maximum(b3, 1e-16)
        ws = w1 + w2 + w3
        for c, (wv, iv) in enumerate(((w1, i1), (w2, i2), (w3, i3))):
            off = c * QPW + g * 16
            wbuf[pl.ds(off, 16)] = wv / ws
            ibuf[pl.ds(off, 16)] = iv

    for c in range(3):
        pltpu.sync_copy(wbuf.at[pl.ds(c * QPW, QPW)],
                        wn_hbm.at[c, pl.ds(base, QPW)])
        pltpu.sync_copy(ibuf.at[pl.ds(c * QPW, QPW)],
                        ix_hbm.at[c, pl.ds(base, QPW)])


def _sc_combine(p_hbm, xw2b_hbm, wn_hbm, ix_hbm, out_hbm,
                wbuf, idx3, rows_v, out_v, sem0, sem1, sem2):
    wid = lax.axis_index("s") * 2 + lax.axis_index("c")
    base = wid * QPW

    for c in range(3):
        pltpu.sync_copy(ix_hbm.at[c, pl.ds(base, QPW)], idx3.at[c])
    sems = (sem0, sem1, sem2)
    cps = [pltpu.async_copy(p_hbm.at[idx3.at[c]],
                            rows_v.at[pl.ds(c * QPW, QPW)], sems[c])
           for c in range(3)]
    pltpu.sync_copy(xw2b_hbm.at[pl.ds(base, QPW)], out_v)
    for c in range(3):
        pltpu.sync_copy(wn_hbm.at[c, pl.ds(base, QPW)],
                        wbuf.at[pl.ds(c * QPW, QPW)])
    for cp in cps:
        cp.wait()

    def qacc(q, _):
        qv = jnp.zeros((16,), jnp.int32) + q
        w0 = plsc.load_gather(wbuf, [qv])
        w1_ = plsc.load_gather(wbuf, [qv + QPW])
        w2_ = plsc.load_gather(wbuf, [qv + 2 * QPW])
        for v in range(DOUT // 16):
            sl = pl.ds(v * 16, 16)
            acc = out_v[q, sl]
            acc = acc + w0 * rows_v[q, sl]
            acc = acc + w1_ * rows_v[QPW + q, sl]
            acc = acc + w2_ * rows_v[2 * QPW + q, sl]
            out_v[q, sl] = acc
        return 0

    lax.fori_loop(0, QPW, qacc, 0, unroll=False)

    pltpu.sync_copy(out_v, out_hbm.at[pl.ds(base, QPW)])


def kernel(images, x, pos, batch, W, b):
    Bq = images.shape[0]
    f32 = jnp.float32
    bf16 = jnp.bfloat16

    W1 = W[:, :C].astype(bf16)
    W2T = jnp.transpose(W[:, C:]).astype(bf16)
    centers = jnp.linspace(1.0 / (D * 2), 1.0 - 1.0 / (D * 2), D).astype(f32)
    cxb = jnp.pad(centers.astype(bf16).astype(f32), (0, 8))
    qx = jnp.pad(centers * centers, (0, 8))
    posx = pos[:, 0]
    posy = pos[:, 1]
    batch_i = batch.astype(jnp.int32)

    mesh = plsc.VectorSubcoreMesh(core_axis_name="c", subcore_axis_name="s")
    scparams = pltpu.CompilerParams(needs_layout_passes=False)

    sel = functools.partial(
        pl.kernel,
        mesh=mesh,
        compiler_params=scparams,
        out_type=(jax.ShapeDtypeStruct((3, M), f32),
                  jax.ShapeDtypeStruct((3, M), jnp.int32)),
        scratch_types=[
            pltpu.VMEM((QPW,), f32),
            pltpu.VMEM((QPW,), f32),
            pltpu.VMEM((QPW,), jnp.int32),
            pltpu.VMEM((64,), f32),
            pltpu.VMEM((64,), f32),
            pltpu.VMEM((T * 16,), f32),
            pltpu.VMEM((T * 16,), f32),
            pltpu.VMEM((T * 16,), f32),
            pltpu.VMEM((T * 16,), f32),
            pltpu.VMEM((T * 16,), f32),
            pltpu.VMEM((T * 16,), f32),
            pltpu.VMEM((32,), f32),
            pltpu.VMEM((3 * QPW,), f32),
            pltpu.VMEM((3 * QPW,), jnp.int32),
        ],
    )(_sc_select)
    wn3, ix3 = sel(posx, posy, batch_i, cxb, qx)

    imgs = images.reshape(Bq, C, HW)
    P = pl.pallas_call(
        _tc_pixproj,
        grid=(Bq,),
        in_specs=[
            pl.BlockSpec((1, C, HW), lambda i: (i, 0, 0)),
            pl.BlockSpec((DOUT, C), lambda i: (0, 0)),
        ],
        out_specs=pl.BlockSpec((HW, DOUT), lambda i: (i, 0)),
        out_shape=jax.ShapeDtypeStruct((Bq * HW, DOUT), f32),
    )(imgs, W1)

    xw2b = pl.pallas_call(
        _tc_xw2b,
        in_specs=[
            pl.BlockSpec(x.shape, lambda: (0, 0)),
            pl.BlockSpec(W2T.shape, lambda: (0, 0)),
            pl.BlockSpec((1, DOUT), lambda: (0, 0)),
        ],
        out_specs=pl.BlockSpec((M, DOUT), lambda: (0, 0)),
        out_shape=jax.ShapeDtypeStruct((M, DOUT), f32),
    )(x.astype(bf16), W2T, b.reshape(1, DOUT))

    comb = functools.partial(
        pl.kernel,
        mesh=mesh,
        compiler_params=scparams,
        out_type=jax.ShapeDtypeStruct((M, DOUT), f32),
        scratch_types=[
            pltpu.VMEM((3 * QPW,), f32),
            pltpu.VMEM((3, QPW), jnp.int32),
            pltpu.VMEM((3 * QPW, DOUT), f32),
            pltpu.VMEM((QPW, DOUT), f32),
            pltpu.SemaphoreType.DMA,
            pltpu.SemaphoreType.DMA,
            pltpu.SemaphoreType.DMA,
        ],
    )(_sc_combine)
    out = comb(P, xw2b, wn3, ix3)

    return (out, pos, batch)
